# Initial kernel scaffold; baseline (speedup 1.0000x reference)
import jax
import jax.numpy as jnp
from jax.experimental import pallas as pl


def kernel(x, edge_index, W1, b1, W2, b2, W3, b3, W_ih, W_hh, b_ih, b_hh, Wc1, bc1, Wc2, bc2, Wr1, br1, Wr2, br2):
    raise NotImplementedError("write your pallas kernel here")


# speedup vs baseline: 10.5114x; 10.5114x over previous
import jax
import jax.numpy as jnp
from jax import lax
from jax.experimental import pallas as pl
from jax.experimental.pallas import tpu as pltpu
from jax.experimental.pallas import tpu_sc as plsc

_N = 100000
_E = 3200000
_HID = 64
_NJ = 33

_B = 128
_EP = 3276800
_ROWS = _EP // _B
_M = 8
_L = 4

_CH = 32
_NC1 = 25
_NC23 = 50

_SW = 8
_NS = 8

_ZR = 6256
_ZQ = 1564
_ZREP = 4
_NACC = 16 * _ZR

_BN = 1088
_GRID = _NACC // _BN

_f32 = jnp.float32


def _sc_mesh():
  return plsc.VectorSubcoreMesh(core_axis_name="c", subcore_axis_name="s")


def _zero_acc(zbuf, acc, sid):
  base = sid * _ZR
  for z in range(_ZREP):
    pltpu.sync_copy(zbuf, acc.at[pl.ds(base + z * _ZQ, _ZQ)])


def _edge_chunks(nchunks, rowbase, src2, dst2, gbuf, sbuf, rows, acc, table,
                 tsem, ssem):
  def chunk(c, carry):
    crow = rowbase + c * _CH
    pltpu.sync_copy(src2.at[pl.ds(crow, _CH)], gbuf)
    pltpu.sync_copy(dst2.at[pl.ds(crow, _CH)], sbuf)
    gd = [None] * _CH
    sd = [None] * _CH
    for k in range(_L):
      gd[k] = pltpu.async_copy(table.at[gbuf.at[k]], rows.at[k % _M], tsem)
    for k in range(_CH):
      kn = k + _L
      if kn < _CH:
        if kn >= _M:
          sd[kn - _M].wait()
        gd[kn] = pltpu.async_copy(table.at[gbuf.at[kn]], rows.at[kn % _M],
                                  tsem)
      gd[k].wait()
      sd[k] = pltpu.async_copy(rows.at[k % _M], acc.at[sbuf.at[k]], ssem,
                               add=True)
    for k in range(max(0, _CH - _M), _CH):
      sd[k].wait()
    return carry

  lax.fori_loop(0, nchunks, chunk, 0)


def _copy_out(acc, out, sid):
  rows = pl.ds(sid * _ZR, _ZR)
  pltpu.sync_copy(acc.at[rows], out.at[rows])


def _deg_body(dst2, zeros_h, ones_h, outa, outb, zbuf, obuf, sbuf, acc, ssem):
  ci = lax.axis_index("c")
  sid = lax.axis_index("s")
  pltpu.sync_copy(zeros_h, zbuf)
  pltpu.sync_copy(ones_h, obuf)
  _zero_acc(zbuf, acc, sid)
  plsc.subcore_barrier()

  rowbase = ci * (_ROWS // 2) + sid * (_ROWS // 32)

  def chunk(c, carry):
    crow = rowbase + c * _CH
    pltpu.sync_copy(dst2.at[pl.ds(crow, _CH)], sbuf)
    sd = [None] * _CH
    for k in range(_CH):
      if k >= _M:
        sd[k - _M].wait()
      sd[k] = pltpu.async_copy(obuf, acc.at[sbuf.at[k]], ssem, add=True)
    for k in range(_CH - _M, _CH):
      sd[k].wait()
    return carry

  lax.fori_loop(0, _NC1, chunk, 0)
  plsc.subcore_barrier()

  @pl.when(ci == 0)
  def _():
    _copy_out(acc, outa, sid)

  @pl.when(ci == 1)
  def _():
    _copy_out(acc, outb, sid)


def _l1_body(src2, dst2, zeros_h, table, outa, outb, zbuf, gbuf, sbuf, rows,
             acc, tsem, ssem):
  ci = lax.axis_index("c")
  sid = lax.axis_index("s")
  pltpu.sync_copy(zeros_h, zbuf)
  _zero_acc(zbuf, acc, sid)
  plsc.subcore_barrier()

  rowbase = ci * (_ROWS // 2) + sid * (_ROWS // 32)
  _edge_chunks(_NC1, rowbase, src2, dst2, gbuf, sbuf, rows, acc, table,
               tsem, ssem)
  plsc.subcore_barrier()

  @pl.when(ci == 0)
  def _():
    _copy_out(acc, outa, sid)

  @pl.when(ci == 1)
  def _():
    _copy_out(acc, outb, sid)


def _l23_body(src2, dst2, zeros_h, m0, m1, m2, m3, m4, m5, m6, m7,
              z0, z1, z2, z3, z4, z5, z6, z7,
              zbuf, gbuf, sbuf, rows, acc, tsem, ssem):
  ci = lax.axis_index("c")
  sid = lax.axis_index("s")
  pltpu.sync_copy(zeros_h, zbuf)
  tables = (m0, m1, m2, m3, m4, m5, m6, m7)
  outs = (z0, z1, z2, z3, z4, z5, z6, z7)
  rowbase = sid * (_ROWS // 16)
  for p in range(4):
    _zero_acc(zbuf, acc, sid)
    plsc.subcore_barrier()

    @pl.when(ci == 0)
    def _():
      _edge_chunks(_NC23, rowbase, src2, dst2, gbuf, sbuf, rows, acc,
                   tables[p], tsem, ssem)

    @pl.when(ci == 1)
    def _():
      _edge_chunks(_NC23, rowbase, src2, dst2, gbuf, sbuf, rows, acc,
                   tables[4 + p], tsem, ssem)

    plsc.subcore_barrier()

    @pl.when(ci == 0)
    def _():
      _copy_out(acc, outs[p], sid)

    @pl.when(ci == 1)
    def _():
      _copy_out(acc, outs[4 + p], sid)

    plsc.subcore_barrier()


def _slice_sds(n):
  return tuple(jax.ShapeDtypeStruct((_NACC, _SW), _f32) for _ in range(n))


def _make_deg():
  return pl.kernel(
      _deg_body,
      out_type=_slice_sds(2),
      mesh=_sc_mesh(),
      compiler_params=pltpu.CompilerParams(use_tc_tiling_on_sc=False),
      scratch_types=[
          pltpu.VMEM((_ZQ, _SW), _f32),
          pltpu.VMEM((_B, _SW), _f32),
          pltpu.VMEM((_CH, _B), jnp.int32),
          pltpu.VMEM_SHARED((_NACC, _SW), _f32),
          pltpu.SemaphoreType.DMA,
      ],
  )


def _make_l1():
  return pl.kernel(
      _l1_body,
      out_type=_slice_sds(2),
      mesh=_sc_mesh(),
      compiler_params=pltpu.CompilerParams(use_tc_tiling_on_sc=False),
      scratch_types=[
          pltpu.VMEM((_ZQ, _SW), _f32),
          pltpu.VMEM((_CH, _B), jnp.int32),
          pltpu.VMEM((_CH, _B), jnp.int32),
          pltpu.VMEM((_M, _B, _SW), _f32),
          pltpu.VMEM_SHARED((_NACC, _SW), _f32),
          pltpu.SemaphoreType.DMA,
          pltpu.SemaphoreType.DMA,
      ],
  )


def _make_l23():
  return pl.kernel(
      _l23_body,
      out_type=_slice_sds(_NS),
      mesh=_sc_mesh(),
      compiler_params=pltpu.CompilerParams(use_tc_tiling_on_sc=False),
      scratch_types=[
          pltpu.VMEM((_ZQ, _SW), _f32),
          pltpu.VMEM((_CH, _B), jnp.int32),
          pltpu.VMEM((_CH, _B), jnp.int32),
          pltpu.VMEM((_M, _B, _SW), _f32),
          pltpu.VMEM_SHARED((_NACC, _SW), _f32),
          pltpu.SemaphoreType.DMA,
          pltpu.SemaphoreType.DMA,
      ],
  )


def _row_spec():
  return pl.BlockSpec((_BN, _SW), lambda i: (i, 0))


def _full_spec(shape):
  return pl.BlockSpec(shape, lambda i: tuple(0 for _ in shape))


def _tca_body(dega, degb, x8, dinv8, q):
  deg = dega[:, 0:1] + degb[:, 0:1] + 1.0
  dinv = lax.rsqrt(jnp.maximum(deg, 1.0))
  d8 = jnp.broadcast_to(dinv, (_BN, _SW))
  dinv8[...] = d8
  q[...] = d8 * x8[...]


def _tca(dega, degb, x8):
  return pl.pallas_call(
      _tca_body,
      grid=(_GRID,),
      in_specs=[_row_spec(), _row_spec(), _row_spec()],
      out_specs=[_row_spec(), _row_spec()],
      out_shape=list(_slice_sds(2)),
  )(dega, degb, x8)


def _tcb_body(z1a, z1b, q, dinv8, w1p, b1, w2, *outs):
  a8 = dinv8[...] * (z1a[...] + z1b[...] + q[...])
  h1 = jnp.maximum(
      jnp.dot(a8, w1p[...], preferred_element_type=_f32) + b1[...], 0.0)
  hs = h1 * dinv8[:, 0:1]
  for p in range(_NS):
    outs[p][...] = jnp.dot(hs, w2[:, _SW * p:_SW * (p + 1)],
                           preferred_element_type=_f32)


def _tcb(z1a, z1b, q, dinv8, w1p, b1, w2):
  return pl.pallas_call(
      _tcb_body,
      grid=(_GRID,),
      in_specs=[_row_spec(), _row_spec(), _row_spec(), _row_spec(),
                _full_spec((_SW, _HID)), _full_spec((1, _HID)),
                _full_spec((_HID, _HID))],
      out_specs=[_row_spec() for _ in range(_NS)],
      out_shape=list(_slice_sds(_NS)),
  )(z1a, z1b, q, dinv8, w1p, b1, w2)


def _tcc_body(*refs):
  zs = refs[0:_NS]
  ms = refs[_NS:2 * _NS]
  dinv8, b2, w3 = refs[2 * _NS:2 * _NS + 3]
  outs = refs[2 * _NS + 3:]
  dcol = dinv8[:, 0:1]
  m3 = None
  for p in range(_NS):
    h2p = jnp.maximum(
        dcol * (zs[p][...] + ms[p][...]) + b2[:, _SW * p:_SW * (p + 1)], 0.0)
    part = jnp.dot(h2p * dcol, w3[_SW * p:_SW * (p + 1), :],
                   preferred_element_type=_f32)
    m3 = part if m3 is None else m3 + part
  for p in range(_NS):
    outs[p][...] = m3[:, _SW * p:_SW * (p + 1)]


def _tcc(zs, ms, dinv8, b2, w3):
  return pl.pallas_call(
      _tcc_body,
      grid=(_GRID,),
      in_specs=[_row_spec() for _ in range(2 * _NS)]
      + [_row_spec(), _full_spec((1, _HID)), _full_spec((_HID, _HID))],
      out_specs=[_row_spec() for _ in range(_NS)],
      out_shape=list(_slice_sds(_NS)),
  )(*zs, *ms, dinv8, b2, w3)


def _tcd_body(*refs):
  zs = refs[0:_NS]
  ms = refs[_NS:2 * _NS]
  dinv8, b3 = refs[2 * _NS:2 * _NS + 2]
  psum = refs[2 * _NS + 2]
  i = pl.program_id(0)
  dcol = dinv8[:, 0:1]
  rowid = i * _BN + lax.broadcasted_iota(jnp.int32, (_BN, 1), 0)
  valid = rowid < _N
  parts = []
  for p in range(_NS):
    h3p = jnp.maximum(
        dcol * (zs[p][...] + ms[p][...]) + b3[:, _SW * p:_SW * (p + 1)], 0.0)
    h3p = jnp.where(valid, h3p, 0.0)
    parts.append(jnp.sum(h3p, axis=0, keepdims=True))

  @pl.when(i == 0)
  def _():
    psum[...] = jnp.zeros((1, _HID), _f32)

  psum[...] += jnp.concatenate(parts, axis=1)


def _tcd(zs, ms, dinv8, b3):
  return pl.pallas_call(
      _tcd_body,
      grid=(_GRID,),
      in_specs=[_row_spec() for _ in range(2 * _NS)]
      + [_row_spec(), _full_spec((1, _HID))],
      out_specs=pl.BlockSpec((1, _HID), lambda i: (0, 0)),
      out_shape=jax.ShapeDtypeStruct((1, _HID), _f32),
  )(*zs, *ms, dinv8, b3)


def _tce_body(psum, wiht, bsum, wc1, bc1, wc2, bc2, wr1, br1, wr2, br2, out):
  pooled = psum[...] / float(_N)
  gates = jnp.dot(pooled, wiht[...], preferred_element_type=_f32) + bsum[...]
  i_g = jax.nn.sigmoid(gates[:, 0:_HID])
  g_g = jnp.tanh(gates[:, 2 * _HID:3 * _HID])
  o_g = jax.nn.sigmoid(gates[:, 3 * _HID:4 * _HID])
  h_l = o_g * jnp.tanh(i_g * g_g)
  j1 = jnp.maximum(
      jnp.dot(h_l, wc1[...], preferred_element_type=_f32) + bc1[...], 0.0)
  joints = jnp.dot(j1, wc2[...], preferred_element_type=_f32) + bc2[...]
  t1 = jnp.maximum(
      jnp.dot(h_l, wr1[...], preferred_element_type=_f32) + br1[...], 0.0)
  torques = jnp.dot(t1, wr2[...], preferred_element_type=_f32) + br2[...]
  out[...] = jnp.concatenate([joints, torques], axis=1)


def _tce(psum, wiht, bsum, wc1, bc1, wc2, bc2, wr1, br1, wr2, br2):
  args = (psum, wiht, bsum, wc1, bc1, wc2, bc2, wr1, br1, wr2, br2)
  specs = [pl.BlockSpec(a.shape, (lambda *_: tuple(0 for _ in range(2))))
           for a in args]
  return pl.pallas_call(
      _tce_body,
      grid=(1,),
      in_specs=specs,
      out_specs=pl.BlockSpec((1, 2 * _NJ * 3), lambda i: (0, 0)),
      out_shape=jax.ShapeDtypeStruct((1, 2 * _NJ * 3), _f32),
  )(*args)


def kernel(x, edge_index, W1, b1, W2, b2, W3, b3, W_ih, W_hh, b_ih, b_hh,
           Wc1, bc1, Wc2, bc2, Wr1, br1, Wr2, br2):
  src = edge_index[0]
  dst = edge_index[1]
  pad = _EP - _E
  src2 = jnp.concatenate([src, jnp.zeros((pad,), jnp.int32)]).reshape(
      _ROWS, _B)
  dst2 = jnp.concatenate([dst, jnp.full((pad,), _N, jnp.int32)]).reshape(
      _ROWS, _B)
  x8 = jnp.zeros((_NACC, _SW), _f32).at[:_N, :3].set(x)
  w1p = jnp.zeros((_SW, _HID), _f32).at[:3].set(W1)
  zeros_h = jnp.zeros((_ZQ, _SW), _f32)
  ones_h = jnp.zeros((_B, _SW), _f32).at[:, 0].set(1.0)

  dega, degb = _make_deg()(dst2, zeros_h, ones_h)
  dinv8, q = _tca(dega, degb, x8)
  z1a, z1b = _make_l1()(src2, dst2, zeros_h, q)
  m2 = _tcb(z1a, z1b, q, dinv8, w1p, b1[None, :], W2)
  z2 = _make_l23()(src2, dst2, zeros_h, *m2)
  m3 = _tcc(z2, m2, dinv8, b2[None, :], W3)
  z3 = _make_l23()(src2, dst2, zeros_h, *m3)
  psum = _tcd(z3, m3, dinv8, b3[None, :])
  out = _tce(psum, W_ih.T, (b_ih + b_hh)[None, :], Wc1, bc1[None, :],
             Wc2, bc2[None, :], Wr1, br1[None, :], Wr2, br2[None, :])
  return out.reshape(2 * _NJ * 3)


# speedup vs baseline: 11.3141x; 1.0764x over previous
import jax
import jax.numpy as jnp
from jax import lax
from jax.experimental import pallas as pl
from jax.experimental.pallas import tpu as pltpu
from jax.experimental.pallas import tpu_sc as plsc

_N = 100000
_E = 3200000
_HID = 64
_NJ = 33

_B = 128
_EP = 3276800
_ROWS = _EP // _B
_M = 16
_L = 8

_CH = 32
_NC1 = 25
_NC23 = 50

_SW = 8
_NS = 8

_ZR = 6256
_ZQ = 1564
_ZREP = 4
_NACC = 16 * _ZR

_BN = 1088
_GRID = _NACC // _BN

_f32 = jnp.float32


def _sc_mesh():
  return plsc.VectorSubcoreMesh(core_axis_name="c", subcore_axis_name="s")


def _zero_acc(zbuf, acc, sid):
  base = sid * _ZR
  for z in range(_ZREP):
    pltpu.sync_copy(zbuf, acc.at[pl.ds(base + z * _ZQ, _ZQ)])


def _edge_chunks(nchunks, rowbase, src2, dst2, gbuf, sbuf, rows, acc, table,
                 tsem, ssem):
  def chunk(c, carry):
    crow = rowbase + c * _CH
    pltpu.sync_copy(src2.at[pl.ds(crow, _CH)], gbuf)
    pltpu.sync_copy(dst2.at[pl.ds(crow, _CH)], sbuf)
    gd = [None] * _CH
    sd = [None] * _CH
    for k in range(_L):
      gd[k] = pltpu.async_copy(table.at[gbuf.at[k]], rows.at[k % _M], tsem)
    for k in range(_CH):
      kn = k + _L
      if kn < _CH:
        if kn >= _M:
          sd[kn - _M].wait()
        gd[kn] = pltpu.async_copy(table.at[gbuf.at[kn]], rows.at[kn % _M],
                                  tsem)
      gd[k].wait()
      sd[k] = pltpu.async_copy(rows.at[k % _M], acc.at[sbuf.at[k]], ssem,
                               add=True)
    for k in range(max(0, _CH - _M), _CH):
      sd[k].wait()
    return carry

  lax.fori_loop(0, nchunks, chunk, 0)


def _copy_out(acc, out, sid):
  rows = pl.ds(sid * _ZR, _ZR)
  pltpu.sync_copy(acc.at[rows], out.at[rows])


def _deg_body(dst2, zeros_h, ones_h, outa, outb, zbuf, obuf, sbuf, acc, ssem):
  ci = lax.axis_index("c")
  sid = lax.axis_index("s")
  pltpu.sync_copy(zeros_h, zbuf)
  pltpu.sync_copy(ones_h, obuf)
  _zero_acc(zbuf, acc, sid)
  plsc.subcore_barrier()

  rowbase = ci * (_ROWS // 2) + sid * (_ROWS // 32)

  def chunk(c, carry):
    crow = rowbase + c * _CH
    pltpu.sync_copy(dst2.at[pl.ds(crow, _CH)], sbuf)
    sd = [None] * _CH
    for k in range(_CH):
      if k >= _M:
        sd[k - _M].wait()
      sd[k] = pltpu.async_copy(obuf, acc.at[sbuf.at[k]], ssem, add=True)
    for k in range(_CH - _M, _CH):
      sd[k].wait()
    return carry

  lax.fori_loop(0, _NC1, chunk, 0)
  plsc.subcore_barrier()

  @pl.when(ci == 0)
  def _():
    _copy_out(acc, outa, sid)

  @pl.when(ci == 1)
  def _():
    _copy_out(acc, outb, sid)


def _l1_body(src2, dst2, zeros_h, table, outa, outb, zbuf, gbuf, sbuf, rows,
             acc, tsem, ssem):
  ci = lax.axis_index("c")
  sid = lax.axis_index("s")
  pltpu.sync_copy(zeros_h, zbuf)
  _zero_acc(zbuf, acc, sid)
  plsc.subcore_barrier()

  rowbase = ci * (_ROWS // 2) + sid * (_ROWS // 32)
  _edge_chunks(_NC1, rowbase, src2, dst2, gbuf, sbuf, rows, acc, table,
               tsem, ssem)
  plsc.subcore_barrier()

  @pl.when(ci == 0)
  def _():
    _copy_out(acc, outa, sid)

  @pl.when(ci == 1)
  def _():
    _copy_out(acc, outb, sid)


def _l23_body(src2, dst2, zeros_h, m0, m1, m2, m3, m4, m5, m6, m7,
              z0, z1, z2, z3, z4, z5, z6, z7,
              zbuf, gbuf, sbuf, rows, acc, tsem, ssem):
  ci = lax.axis_index("c")
  sid = lax.axis_index("s")
  pltpu.sync_copy(zeros_h, zbuf)
  tables = (m0, m1, m2, m3, m4, m5, m6, m7)
  outs = (z0, z1, z2, z3, z4, z5, z6, z7)
  rowbase = sid * (_ROWS // 16)
  for p in range(4):
    _zero_acc(zbuf, acc, sid)
    plsc.subcore_barrier()

    @pl.when(ci == 0)
    def _():
      _edge_chunks(_NC23, rowbase, src2, dst2, gbuf, sbuf, rows, acc,
                   tables[p], tsem, ssem)

    @pl.when(ci == 1)
    def _():
      _edge_chunks(_NC23, rowbase, src2, dst2, gbuf, sbuf, rows, acc,
                   tables[4 + p], tsem, ssem)

    plsc.subcore_barrier()

    @pl.when(ci == 0)
    def _():
      _copy_out(acc, outs[p], sid)

    @pl.when(ci == 1)
    def _():
      _copy_out(acc, outs[4 + p], sid)

    plsc.subcore_barrier()


def _slice_sds(n):
  return tuple(jax.ShapeDtypeStruct((_NACC, _SW), _f32) for _ in range(n))


def _make_deg():
  return pl.kernel(
      _deg_body,
      out_type=_slice_sds(2),
      mesh=_sc_mesh(),
      compiler_params=pltpu.CompilerParams(use_tc_tiling_on_sc=False),
      scratch_types=[
          pltpu.VMEM((_ZQ, _SW), _f32),
          pltpu.VMEM((_B, _SW), _f32),
          pltpu.VMEM((_CH, _B), jnp.int32),
          pltpu.VMEM_SHARED((_NACC, _SW), _f32),
          pltpu.SemaphoreType.DMA,
      ],
  )


def _make_l1():
  return pl.kernel(
      _l1_body,
      out_type=_slice_sds(2),
      mesh=_sc_mesh(),
      compiler_params=pltpu.CompilerParams(use_tc_tiling_on_sc=False),
      scratch_types=[
          pltpu.VMEM((_ZQ, _SW), _f32),
          pltpu.VMEM((_CH, _B), jnp.int32),
          pltpu.VMEM((_CH, _B), jnp.int32),
          pltpu.VMEM((_M, _B, _SW), _f32),
          pltpu.VMEM_SHARED((_NACC, _SW), _f32),
          pltpu.SemaphoreType.DMA,
          pltpu.SemaphoreType.DMA,
      ],
  )


def _make_l23():
  return pl.kernel(
      _l23_body,
      out_type=_slice_sds(_NS),
      mesh=_sc_mesh(),
      compiler_params=pltpu.CompilerParams(use_tc_tiling_on_sc=False),
      scratch_types=[
          pltpu.VMEM((_ZQ, _SW), _f32),
          pltpu.VMEM((_CH, _B), jnp.int32),
          pltpu.VMEM((_CH, _B), jnp.int32),
          pltpu.VMEM((_M, _B, _SW), _f32),
          pltpu.VMEM_SHARED((_NACC, _SW), _f32),
          pltpu.SemaphoreType.DMA,
          pltpu.SemaphoreType.DMA,
      ],
  )


def _row_spec():
  return pl.BlockSpec((_BN, _SW), lambda i: (i, 0))


def _full_spec(shape):
  return pl.BlockSpec(shape, lambda i: tuple(0 for _ in shape))


def _tca_body(dega, degb, x8, dinv8, q):
  deg = dega[:, 0:1] + degb[:, 0:1] + 1.0
  dinv = lax.rsqrt(jnp.maximum(deg, 1.0))
  d8 = jnp.broadcast_to(dinv, (_BN, _SW))
  dinv8[...] = d8
  q[...] = d8 * x8[...]


def _tca(dega, degb, x8):
  return pl.pallas_call(
      _tca_body,
      grid=(_GRID,),
      in_specs=[_row_spec(), _row_spec(), _row_spec()],
      out_specs=[_row_spec(), _row_spec()],
      out_shape=list(_slice_sds(2)),
  )(dega, degb, x8)


def _tcb_body(z1a, z1b, q, dinv8, w1p, b1, w2, *outs):
  a8 = dinv8[...] * (z1a[...] + z1b[...] + q[...])
  h1 = jnp.maximum(
      jnp.dot(a8, w1p[...], preferred_element_type=_f32) + b1[...], 0.0)
  hs = h1 * dinv8[:, 0:1]
  for p in range(_NS):
    outs[p][...] = jnp.dot(hs, w2[:, _SW * p:_SW * (p + 1)],
                           preferred_element_type=_f32)


def _tcb(z1a, z1b, q, dinv8, w1p, b1, w2):
  return pl.pallas_call(
      _tcb_body,
      grid=(_GRID,),
      in_specs=[_row_spec(), _row_spec(), _row_spec(), _row_spec(),
                _full_spec((_SW, _HID)), _full_spec((1, _HID)),
                _full_spec((_HID, _HID))],
      out_specs=[_row_spec() for _ in range(_NS)],
      out_shape=list(_slice_sds(_NS)),
  )(z1a, z1b, q, dinv8, w1p, b1, w2)


def _tcc_body(*refs):
  zs = refs[0:_NS]
  ms = refs[_NS:2 * _NS]
  dinv8, b2, w3 = refs[2 * _NS:2 * _NS + 3]
  outs = refs[2 * _NS + 3:]
  dcol = dinv8[:, 0:1]
  m3 = None
  for p in range(_NS):
    h2p = jnp.maximum(
        dcol * (zs[p][...] + ms[p][...]) + b2[:, _SW * p:_SW * (p + 1)], 0.0)
    part = jnp.dot(h2p * dcol, w3[_SW * p:_SW * (p + 1), :],
                   preferred_element_type=_f32)
    m3 = part if m3 is None else m3 + part
  for p in range(_NS):
    outs[p][...] = m3[:, _SW * p:_SW * (p + 1)]


def _tcc(zs, ms, dinv8, b2, w3):
  return pl.pallas_call(
      _tcc_body,
      grid=(_GRID,),
      in_specs=[_row_spec() for _ in range(2 * _NS)]
      + [_row_spec(), _full_spec((1, _HID)), _full_spec((_HID, _HID))],
      out_specs=[_row_spec() for _ in range(_NS)],
      out_shape=list(_slice_sds(_NS)),
  )(*zs, *ms, dinv8, b2, w3)


def _tcd_body(*refs):
  zs = refs[0:_NS]
  ms = refs[_NS:2 * _NS]
  dinv8, b3 = refs[2 * _NS:2 * _NS + 2]
  psum = refs[2 * _NS + 2]
  i = pl.program_id(0)
  dcol = dinv8[:, 0:1]
  rowid = i * _BN + lax.broadcasted_iota(jnp.int32, (_BN, 1), 0)
  valid = rowid < _N
  parts = []
  for p in range(_NS):
    h3p = jnp.maximum(
        dcol * (zs[p][...] + ms[p][...]) + b3[:, _SW * p:_SW * (p + 1)], 0.0)
    h3p = jnp.where(valid, h3p, 0.0)
    parts.append(jnp.sum(h3p, axis=0, keepdims=True))

  @pl.when(i == 0)
  def _():
    psum[...] = jnp.zeros((1, _HID), _f32)

  psum[...] += jnp.concatenate(parts, axis=1)


def _tcd(zs, ms, dinv8, b3):
  return pl.pallas_call(
      _tcd_body,
      grid=(_GRID,),
      in_specs=[_row_spec() for _ in range(2 * _NS)]
      + [_row_spec(), _full_spec((1, _HID))],
      out_specs=pl.BlockSpec((1, _HID), lambda i: (0, 0)),
      out_shape=jax.ShapeDtypeStruct((1, _HID), _f32),
  )(*zs, *ms, dinv8, b3)


def _tce_body(psum, wiht, bsum, wc1, bc1, wc2, bc2, wr1, br1, wr2, br2, out):
  pooled = psum[...] / float(_N)
  gates = jnp.dot(pooled, wiht[...], preferred_element_type=_f32) + bsum[...]
  i_g = jax.nn.sigmoid(gates[:, 0:_HID])
  g_g = jnp.tanh(gates[:, 2 * _HID:3 * _HID])
  o_g = jax.nn.sigmoid(gates[:, 3 * _HID:4 * _HID])
  h_l = o_g * jnp.tanh(i_g * g_g)
  j1 = jnp.maximum(
      jnp.dot(h_l, wc1[...], preferred_element_type=_f32) + bc1[...], 0.0)
  joints = jnp.dot(j1, wc2[...], preferred_element_type=_f32) + bc2[...]
  t1 = jnp.maximum(
      jnp.dot(h_l, wr1[...], preferred_element_type=_f32) + br1[...], 0.0)
  torques = jnp.dot(t1, wr2[...], preferred_element_type=_f32) + br2[...]
  out[...] = jnp.concatenate([joints, torques], axis=1)


def _tce(psum, wiht, bsum, wc1, bc1, wc2, bc2, wr1, br1, wr2, br2):
  args = (psum, wiht, bsum, wc1, bc1, wc2, bc2, wr1, br1, wr2, br2)
  specs = [pl.BlockSpec(a.shape, (lambda *_: tuple(0 for _ in range(2))))
           for a in args]
  return pl.pallas_call(
      _tce_body,
      grid=(1,),
      in_specs=specs,
      out_specs=pl.BlockSpec((1, 2 * _NJ * 3), lambda i: (0, 0)),
      out_shape=jax.ShapeDtypeStruct((1, 2 * _NJ * 3), _f32),
  )(*args)


def kernel(x, edge_index, W1, b1, W2, b2, W3, b3, W_ih, W_hh, b_ih, b_hh,
           Wc1, bc1, Wc2, bc2, Wr1, br1, Wr2, br2):
  src = edge_index[0]
  dst = edge_index[1]
  pad = _EP - _E
  src2 = jnp.concatenate([src, jnp.zeros((pad,), jnp.int32)]).reshape(
      _ROWS, _B)
  dst2 = jnp.concatenate([dst, jnp.full((pad,), _N, jnp.int32)]).reshape(
      _ROWS, _B)
  x8 = jnp.zeros((_NACC, _SW), _f32).at[:_N, :3].set(x)
  w1p = jnp.zeros((_SW, _HID), _f32).at[:3].set(W1)
  zeros_h = jnp.zeros((_ZQ, _SW), _f32)
  ones_h = jnp.zeros((_B, _SW), _f32).at[:, 0].set(1.0)

  dega, degb = _make_deg()(dst2, zeros_h, ones_h)
  dinv8, q = _tca(dega, degb, x8)
  z1a, z1b = _make_l1()(src2, dst2, zeros_h, q)
  m2 = _tcb(z1a, z1b, q, dinv8, w1p, b1[None, :], W2)
  z2 = _make_l23()(src2, dst2, zeros_h, *m2)
  m3 = _tcc(z2, m2, dinv8, b2[None, :], W3)
  z3 = _make_l23()(src2, dst2, zeros_h, *m3)
  psum = _tcd(z3, m3, dinv8, b3[None, :])
  out = _tce(psum, W_ih.T, (b_ih + b_hh)[None, :], Wc1, bc1[None, :],
             Wc2, bc2[None, :], Wr1, br1[None, :], Wr2, br2[None, :])
  return out.reshape(2 * _NJ * 3)
